# baseline (speedup 1.0000x reference)
import functools

import jax
import jax.numpy as jnp
from jax import lax
from jax.experimental import pallas as pl
from jax.experimental.pallas import tpu as pltpu
from jax.experimental.pallas import tpu_sc as plsc

NC = 2
NS = 16
NW = NC * NS
L = 16


@functools.lru_cache(maxsize=None)
def _build(B, D, K, N, DW):
    assert B % NW == 0, B
    assert D % L == 0 and DW >= D, D
    assert K % 2 == 0 and (K // 2) <= 128 and K % 8 == 0 and K >= L, K
    bpw = B // NW
    kh = K // 2
    ngroups = (K + L - 1) // L
    kpad = ngroups * L
    ndc = D // L

    mesh = plsc.VectorSubcoreMesh(core_axis_name="c", subcore_axis_name="s")

    @functools.partial(
        pl.kernel,
        mesh=mesh,
        compiler_params=pltpu.CompilerParams(use_tc_tiling_on_sc=True),
        out_type=jax.ShapeDtypeStruct((B, kpad), jnp.float32),
        scratch_types=[
            pltpu.VMEM((bpw, 2, kh), jnp.int32),
            pltpu.VMEM((bpw, D), jnp.float32),
            pltpu.VMEM((2, kpad, DW), jnp.float32),
            pltpu.VMEM((2, kpad), jnp.float32),
            pltpu.SemaphoreType.DMA,
            pltpu.SemaphoreType.DMA,
            pltpu.SemaphoreType.DMA,
            pltpu.SemaphoreType.DMA,
        ],
    )
    def sc_kernel(h_hbm, ids_hbm, w_hbm, out_hbm, idx_v, h_v, emb, out_v,
                  sem0, sem1, semo0, semo1):
        wid = lax.axis_index("s") * NC + lax.axis_index("c")
        base = wid * bpw

        pltpu.sync_copy(ids_hbm.at[pl.ds(base, bpw)], idx_v)
        pltpu.sync_copy(h_hbm.at[pl.ds(base, bpw)], h_v)

        sems = (sem0, sem1)
        semos = (semo0, semo1)

        def fire_out(b, slot):
            pltpu.async_copy(out_v.at[slot], out_hbm.at[base + b],
                             semos[slot])

        def drain_out(slot):
            pltpu.make_async_copy(out_hbm.at[0], out_v.at[slot],
                                  semos[slot]).wait()

        def fire(b, slot):
            pltpu.async_copy(w_hbm.at[idx_v.at[b, 0]],
                             emb.at[slot, pl.ds(0, kh)], sems[slot])
            pltpu.async_copy(w_hbm.at[idx_v.at[b, 1]],
                             emb.at[slot, pl.ds(kh, kh)], sems[slot])

        def drain(slot):
            pltpu.make_async_copy(w_hbm.at[pl.ds(0, K)],
                                  emb.at[slot, pl.ds(0, K)],
                                  sems[slot]).wait()

        lane = lax.iota(jnp.int32, L)
        perms = [lane ^ (1 << i) for i in range(4)]
        masks = [(lane & (1 << i)) == 0 for i in range(4)]

        dnums = lax.GatherDimensionNumbers(
            offset_dims=(), collapsed_slice_dims=(0,), start_index_map=(0,))

        def shuffle(x, perm):
            return lax.gather(x, perm[:, None], dimension_numbers=dnums,
                              slice_sizes=(1,),
                              mode=lax.GatherScatterMode.PROMISE_IN_BOUNDS)

        def combine(x, y, lvl):
            a = jnp.where(masks[lvl], x, y)
            b = jnp.where(masks[lvl], y, x)
            return a + shuffle(b, perms[lvl])

        def compute_row(b, slot):
            hc = [h_v[b, pl.ds(c * L, L)] for c in range(ndc)]

            def group(g, carry):
                kb = pl.multiple_of(g * L, L)
                ps = []
                for j in range(L):
                    p = hc[0] * emb[slot, kb + j, pl.ds(0, L)]
                    for c in range(1, ndc):
                        p = p + hc[c] * emb[slot, kb + j, pl.ds(c * L, L)]
                    ps.append(p)
                for lvl in range(4):
                    ps = [combine(ps[2 * i], ps[2 * i + 1], lvl)
                          for i in range(len(ps) // 2)]
                out_v[slot, pl.ds(kb, L)] = ps[0]
                return carry

            lax.fori_loop(0, ngroups, group, 0)

        fire(0, 0)

        def body(t, carry):
            b = 2 * t
            fire(b + 1, 1)
            drain(0)

            @pl.when(t > 0)
            def _():
                drain_out(0)

            compute_row(b, 0)
            fire_out(b, 0)

            @pl.when(t < bpw // 2 - 1)
            def _():
                fire(b + 2, 0)

            drain(1)

            @pl.when(t > 0)
            def _():
                drain_out(1)

            compute_row(b + 1, 1)
            fire_out(b + 1, 1)
            return carry

        lax.fori_loop(0, bpw // 2, body, 0)
        drain_out(0)
        drain_out(1)

    return sc_kernel


def kernel(h, cand_ids, W):
    B, D = h.shape
    K = cand_ids.shape[1]
    N = W.shape[0]
    ids3 = cand_ids.astype(jnp.int32).reshape(B, 2, K // 2)
    wp = jnp.pad(W, ((0, 0), (0, 128 - D)))
    return _build(B, D, K, N, 128)(h, ids3, wp)[:, :K]
